# Initial kernel scaffold; baseline (speedup 1.0000x reference)
import jax
import jax.numpy as jnp
from jax.experimental import pallas as pl


def kernel(xyz1, xyz2, points1, points2, W1, b1, g1, be1, W2, b2, g2, be2):
    raise NotImplementedError("write your pallas kernel here")


# speedup vs baseline: 9.1494x; 9.1494x over previous
import functools

import jax
import jax.numpy as jnp
from jax.experimental import pallas as pl


def _knn_interp_kernel(xyz1t_ref, xyz2_ref, p2_ref, out_ref):
    q = xyz1t_ref[0]
    k = xyz2_ref[0]
    p2 = p2_ref[0]
    N = q.shape[0]
    S = k.shape[1]
    qb = q.astype(jnp.bfloat16).astype(jnp.float32)
    kb = k.astype(jnp.bfloat16).astype(jnp.float32)
    qk = qb[:, 0:1] * kb[0:1, :]
    qk = qk + qb[:, 1:2] * kb[1:2, :]
    qk = qk + qb[:, 2:3] * kb[2:3, :]
    n1 = q[:, 0:1] * q[:, 0:1]
    n1 = n1 + q[:, 1:2] * q[:, 1:2]
    n1 = n1 + q[:, 2:3] * q[:, 2:3]
    n2 = k[0:1, :] * k[0:1, :]
    n2 = n2 + k[1:2, :] * k[1:2, :]
    n2 = n2 + k[2:3, :] * k[2:3, :]
    d = -2.0 * qk
    d = d + n1
    d = d + n2
    lane = jax.lax.broadcasted_iota(jnp.int32, (N, S), 1)
    mvs = []
    idxs = []
    for _ in range(3):
        mv = jnp.min(d, axis=1, keepdims=True)
        idx = jnp.min(jnp.where(d == mv, lane, S), axis=1, keepdims=True)
        mvs.append(mv)
        idxs.append(idx)
        d = jnp.where(lane == idx, jnp.inf, d)
    r = [1.0 / (mv + 1e-8) for mv in mvs]
    norm = r[0] + r[1] + r[2]
    wmat = jnp.zeros((N, S), dtype=jnp.float32)
    for kk in range(3):
        wmat = wmat + jnp.where(lane == idxs[kk], r[kk] / norm, 0.0)
    interp = jax.lax.dot_general(
        wmat, p2, (((1,), (1,)), ((), ())),
        preferred_element_type=jnp.float32,
        precision=jax.lax.Precision.HIGHEST)
    out_ref[:, 0, 0, :] = interp


def _mm_bn_relu_kernel(w_ref, x_ref, b_ref, g_ref, be_ref, out_ref):
    x1 = jax.lax.dot_general(
        w_ref[...], x_ref[...], (((1,), (0,)), ((), ())),
        preferred_element_type=jnp.float32)
    x1 = x1 + b_ref[...]
    bl = x1.shape[1]
    m = jnp.sum(x1, axis=1, keepdims=True) / bl
    xc = x1 - m
    v = jnp.sum(xc * xc, axis=1, keepdims=True) / bl
    xh = xc * jax.lax.rsqrt(v + 1e-5)
    y = g_ref[...] * xh + be_ref[...]
    out_ref[...] = jnp.maximum(y, 0.0)


def kernel(xyz1, xyz2, points1, points2, W1, b1, g1, be1, W2, b2, g2, be2):
    B, _, N = xyz1.shape
    S = xyz2.shape[2]
    D = points2.shape[1]
    c1 = W1.shape[0]
    c2 = W2.shape[0]
    BL = B * D

    xyz1t = jnp.transpose(xyz1, (0, 2, 1))

    interp_t = pl.pallas_call(
        _knn_interp_kernel,
        grid=(B,),
        in_specs=[
            pl.BlockSpec((1, N, 3), lambda b: (b, 0, 0)),
            pl.BlockSpec((1, 3, S), lambda b: (b, 0, 0)),
            pl.BlockSpec((1, D, S), lambda b: (b, 0, 0)),
        ],
        out_specs=pl.BlockSpec((N, 1, 1, D), lambda b: (0, b, 0, 0)),
        out_shape=jax.ShapeDtypeStruct((N, B, 1, D), jnp.float32),
    )(xyz1t, xyz2, points2)

    p1t = jnp.transpose(points1, (2, 0, 1)).reshape(N, BL)
    np_big = jnp.concatenate([p1t, interp_t.reshape(N, BL)], axis=0)

    def mm_stage(W, b, g, be, x, rows, row_chunk):
        nblk = rows // row_chunk
        cdim = W.shape[1]
        return pl.pallas_call(
            _mm_bn_relu_kernel,
            grid=(nblk,),
            in_specs=[
                pl.BlockSpec((row_chunk, cdim), lambda r: (r, 0)),
                pl.BlockSpec((cdim, BL), lambda r: (0, 0)),
                pl.BlockSpec((row_chunk, 1), lambda r: (r, 0)),
                pl.BlockSpec((row_chunk, 1), lambda r: (r, 0)),
                pl.BlockSpec((row_chunk, 1), lambda r: (r, 0)),
            ],
            out_specs=pl.BlockSpec((row_chunk, BL), lambda r: (r, 0)),
            out_shape=jax.ShapeDtypeStruct((rows, BL), jnp.float32),
        )(W, x, b.reshape(rows, 1), g.reshape(rows, 1), be.reshape(rows, 1))

    y1 = mm_stage(W1, b1, g1, be1, np_big, c1, 256)
    y2 = mm_stage(W2, b2, g2, be2, y1, c2, 256)

    return jnp.transpose(y2.reshape(c2, B, D), (1, 0, 2))
